# Initial kernel scaffold; baseline (speedup 1.0000x reference)
import jax
import jax.numpy as jnp
from jax.experimental import pallas as pl


def kernel(probs, log_beam_prob_prev):
    raise NotImplementedError("write your pallas kernel here")


# speedup vs baseline: 3.6763x; 3.6763x over previous
import functools

import jax
import jax.numpy as jnp
from jax import lax
from jax.experimental import pallas as pl
from jax.experimental.pallas import tpu as pltpu
from jax.experimental.pallas import tpu_sc as plsc

BW = 16
S = 400
NEGF = float("-inf")
BIGI = 2**31 - 1


def _k1_body(probs_ref, prev_ref, out_ref):
    x = probs_ref[...]
    m = jnp.max(x, axis=2)
    out_ref[...] = jnp.log(m) + prev_ref[...]


def _k2a_body(s_ref, prevT_ref, r2_ref, ps_ref, ba_ref, *, B, P, N):
    s = s_ref[...]
    wio = lax.broadcasted_iota(jnp.int32, (B, BW, P), 1)
    jio = lax.broadcasted_iota(jnp.int32, (B, BW, P), 2)
    pio = wio * P + jio
    prevT = prevT_ref[...]
    w16 = lax.broadcasted_iota(jnp.int32, (B, BW), 1)
    bcol = lax.broadcasted_iota(jnp.int32, (B, 1), 0)
    R2 = jnp.zeros((B, BW), jnp.int32)
    PS = jnp.zeros((B, BW), jnp.float32)
    BA = jnp.zeros((B, BW), jnp.int32)
    for k in range(BW):
        t = jnp.max(s, axis=2)
        m = jnp.max(t, axis=1, keepdims=True)
        cand = jnp.where(s == m[:, :, None], pio, BIGI)
        c2 = jnp.min(cand, axis=2)
        pid = jnp.min(c2, axis=1, keepdims=True)
        w = pid // P
        j = pid - w * P
        prevk = jnp.sum(jnp.where(w16 == w, prevT, 0.0), axis=1, keepdims=True)
        colm = w16 == k
        R2 = jnp.where(colm, (w * 32 + bcol) * P + j, R2)
        PS = jnp.where(colm, prevk, PS)
        BA = jnp.where(colm, w * N + j * S, BA)
        s = jnp.where(pio == pid[:, :, None], NEGF, s)
    r2_ref[...] = R2
    ps_ref[...] = PS
    ba_ref[...] = BA


def _k2c_body(g_ref, ps_ref, ba_ref, sel_ref, logp_ref, bbi_ref, *, B, N):
    g = g_ref[...]
    s = jnp.log(g) + ps_ref[...][:, :, None]
    cio = lax.broadcasted_iota(jnp.int32, (B, BW, S), 2)
    gflat = ba_ref[...][:, :, None] + cio
    w16 = lax.broadcasted_iota(jnp.int32, (B, BW), 1)
    bcol = lax.broadcasted_iota(jnp.int32, (B, 1), 0)
    SEL = jnp.zeros((B, BW), jnp.int32)
    LOGP = jnp.zeros((B, BW), jnp.float32)
    BBI = jnp.zeros((B, BW), jnp.int32)
    for k in range(BW):
        t = jnp.max(s, axis=2)
        m = jnp.max(t, axis=1, keepdims=True)
        cand = jnp.where(s == m[:, :, None], gflat, BIGI)
        c2 = jnp.min(cand, axis=2)
        wf = jnp.min(c2, axis=1, keepdims=True)
        par = wf // N
        colm = w16 == k
        SEL = jnp.where(colm, wf - par * N, SEL)
        LOGP = jnp.where(colm, m, LOGP)
        BBI = jnp.where(colm, bcol + par * B, BBI)
        s = jnp.where(gflat == wf[:, :, None], NEGF, s)
    sel_ref[...] = SEL
    logp_ref[...] = LOGP
    bbi_ref[...] = BBI


def _sc_gather(table, idx):
    NR = idx.shape[0]
    D = table.shape[1]
    NW = 32
    bpw = NR // NW
    mesh = plsc.VectorSubcoreMesh(core_axis_name="c", subcore_axis_name="s")

    @functools.partial(
        pl.kernel, mesh=mesh,
        out_type=jax.ShapeDtypeStruct((NR, D), jnp.float32),
        compiler_params=pltpu.CompilerParams(use_tc_tiling_on_sc=False),
        scratch_types=[
            pltpu.VMEM((bpw,), jnp.int32),
            pltpu.VMEM((bpw, D), jnp.float32),
            pltpu.SemaphoreType.DMA,
        ],
    )
    def k(table_hbm, idx_hbm, out_hbm, idx_v, rows_v, sem):
        wid = lax.axis_index("s") * 2 + lax.axis_index("c")
        base = wid * bpw
        pltpu.sync_copy(idx_hbm.at[pl.ds(base, bpw)], idx_v)
        pltpu.async_copy(table_hbm.at[idx_v], rows_v, sem).wait()
        pltpu.sync_copy(rows_v, out_hbm.at[pl.ds(base, bpw)])

    return k(table, idx)


def kernel(probs, log_beam_prob_prev):
    A, N = probs.shape
    B = A // BW
    P = N // S
    RB = 8

    probs3 = probs.reshape(A, P, S)
    prev2 = log_beam_prob_prev.reshape(A, 1)

    smax = pl.pallas_call(
        _k1_body,
        grid=(A // RB,),
        in_specs=[
            pl.BlockSpec((RB, P, S), lambda i: (i, 0, 0)),
            pl.BlockSpec((RB, 1), lambda i: (i, 0)),
        ],
        out_specs=pl.BlockSpec((RB, P), lambda i: (i, 0)),
        out_shape=jax.ShapeDtypeStruct((A, P), jnp.float32),
    )(probs3, prev2)

    smaxT = smax.reshape(BW, B, P).transpose(1, 0, 2)
    prevT = log_beam_prob_prev.reshape(BW, B).T

    r2, ps, ba = pl.pallas_call(
        functools.partial(_k2a_body, B=B, P=P, N=N),
        out_shape=(
            jax.ShapeDtypeStruct((B, BW), jnp.int32),
            jax.ShapeDtypeStruct((B, BW), jnp.float32),
            jax.ShapeDtypeStruct((B, BW), jnp.int32),
        ),
    )(smaxT, prevT)

    g = _sc_gather(probs.reshape(A * P, S), r2.reshape(-1))

    sel, logp, bbi = pl.pallas_call(
        functools.partial(_k2c_body, B=B, N=N),
        out_shape=(
            jax.ShapeDtypeStruct((B, BW), jnp.int32),
            jax.ShapeDtypeStruct((B, BW), jnp.float32),
            jax.ShapeDtypeStruct((B, BW), jnp.int32),
        ),
    )(g.reshape(B, BW, S), ps, ba)

    return (sel.T.reshape(-1), logp.T.reshape(-1), bbi.T.reshape(-1))


# speedup vs baseline: 4.2306x; 1.1508x over previous
import functools

import jax
import jax.numpy as jnp
from jax import lax
from jax.experimental import pallas as pl
from jax.experimental.pallas import tpu as pltpu
from jax.experimental.pallas import tpu_sc as plsc

BW = 16
S = 400
NEGF = float("-inf")
BIGI = 2**31 - 1


def _k1_body(probs_ref, prev_ref, out_ref, *, P):
    x = probs_ref[...]
    m = jnp.max(x.reshape(x.shape[0], P, S), axis=2)
    out_ref[...] = jnp.log(m) + prev_ref[...]


def _k2a_body(s_ref, prevT_ref, row_ref, col_ref, ps_ref, ba_ref, *, B, P, N):
    s = s_ref[...]
    wio = lax.broadcasted_iota(jnp.int32, (B, BW, P), 1)
    jio = lax.broadcasted_iota(jnp.int32, (B, BW, P), 2)
    pio = wio * P + jio
    prevT = prevT_ref[...]
    w16 = lax.broadcasted_iota(jnp.int32, (B, BW), 1)
    bcol = lax.broadcasted_iota(jnp.int32, (B, 1), 0)
    ROW = jnp.zeros((B, BW), jnp.int32)
    COL = jnp.zeros((B, BW), jnp.int32)
    PS = jnp.zeros((B, BW), jnp.float32)
    BA = jnp.zeros((B, BW), jnp.int32)
    for k in range(BW):
        t = jnp.max(s, axis=2)
        m = jnp.max(t, axis=1, keepdims=True)
        cand = jnp.where(s == m[:, :, None], pio, BIGI)
        c2 = jnp.min(cand, axis=2)
        pid = jnp.min(c2, axis=1, keepdims=True)
        w = pid // P
        j = pid - w * P
        prevk = jnp.sum(jnp.where(w16 == w, prevT, 0.0), axis=1, keepdims=True)
        colm = w16 == k
        ROW = jnp.where(colm, w * 32 + bcol, ROW)
        COL = jnp.where(colm, j * S, COL)
        PS = jnp.where(colm, prevk, PS)
        BA = jnp.where(colm, w * N + j * S, BA)
        s = jnp.where(pio == pid[:, :, None], NEGF, s)
    row_ref[...] = ROW
    col_ref[...] = COL
    ps_ref[...] = PS
    ba_ref[...] = BA


def _k2c_body(g_ref, ps_ref, ba_ref, sel_ref, logp_ref, bbi_ref, *, B, N):
    g = g_ref[...]
    s = jnp.log(g) + ps_ref[...][:, :, None]
    cio = lax.broadcasted_iota(jnp.int32, (B, BW, S), 2)
    gflat = ba_ref[...][:, :, None] + cio
    w16 = lax.broadcasted_iota(jnp.int32, (B, BW), 1)
    bcol = lax.broadcasted_iota(jnp.int32, (B, 1), 0)
    SEL = jnp.zeros((B, BW), jnp.int32)
    LOGP = jnp.zeros((B, BW), jnp.float32)
    BBI = jnp.zeros((B, BW), jnp.int32)
    for k in range(BW):
        t = jnp.max(s, axis=2)
        m = jnp.max(t, axis=1, keepdims=True)
        cand = jnp.where(s == m[:, :, None], gflat, BIGI)
        c2 = jnp.min(cand, axis=2)
        wf = jnp.min(c2, axis=1, keepdims=True)
        par = wf // N
        colm = w16 == k
        SEL = jnp.where(colm, wf - par * N, SEL)
        LOGP = jnp.where(colm, m, LOGP)
        BBI = jnp.where(colm, bcol + par * B, BBI)
        s = jnp.where(gflat == wf[:, :, None], NEGF, s)
    sel_ref[...] = SEL
    logp_ref[...] = LOGP
    bbi_ref[...] = BBI


def _sc_gather(probs, rows, cols):
    NR = rows.shape[0]
    NC = 2
    bpc = NR // NC
    mesh = plsc.ScalarSubcoreMesh(axis_name="c", num_cores=NC)

    @functools.partial(
        pl.kernel, mesh=mesh,
        out_type=jax.ShapeDtypeStruct((NR, S), jnp.float32),
        compiler_params=pltpu.CompilerParams(use_tc_tiling_on_sc=False),
        scratch_types=[
            pltpu.SMEM((bpc,), jnp.int32),
            pltpu.SMEM((bpc,), jnp.int32),
            pltpu.SemaphoreType.DMA,
        ],
    )
    def k(probs_hbm, rows_hbm, cols_hbm, out_hbm, row_s, col_s, sem):
        base = lax.axis_index("c") * bpc
        pltpu.sync_copy(rows_hbm.at[pl.ds(base, bpc)], row_s)
        pltpu.sync_copy(cols_hbm.at[pl.ds(base, bpc)], col_s)
        copies = []
        for i in range(bpc):
            c0 = pl.multiple_of(col_s[i], 8)
            copies.append(pltpu.async_copy(
                probs_hbm.at[row_s[i], pl.ds(c0, S)], out_hbm.at[base + i], sem))
        for c in copies:
            c.wait()

    return k(probs, rows, cols)


def kernel(probs, log_beam_prob_prev):
    A, N = probs.shape
    B = A // BW
    P = N // S
    RB = 8

    prev2 = log_beam_prob_prev.reshape(A, 1)

    smax = pl.pallas_call(
        functools.partial(_k1_body, P=P),
        grid=(A // RB,),
        in_specs=[
            pl.BlockSpec((RB, N), lambda i: (i, 0)),
            pl.BlockSpec((RB, 1), lambda i: (i, 0)),
        ],
        out_specs=pl.BlockSpec((RB, P), lambda i: (i, 0)),
        out_shape=jax.ShapeDtypeStruct((A, P), jnp.float32),
    )(probs, prev2)

    smaxT = smax.reshape(BW, B, P).transpose(1, 0, 2)
    prevT = log_beam_prob_prev.reshape(BW, B).T

    row, col, ps, ba = pl.pallas_call(
        functools.partial(_k2a_body, B=B, P=P, N=N),
        out_shape=(
            jax.ShapeDtypeStruct((B, BW), jnp.int32),
            jax.ShapeDtypeStruct((B, BW), jnp.int32),
            jax.ShapeDtypeStruct((B, BW), jnp.float32),
            jax.ShapeDtypeStruct((B, BW), jnp.int32),
        ),
    )(smaxT, prevT)

    g = _sc_gather(probs, row.reshape(-1), col.reshape(-1))

    sel, logp, bbi = pl.pallas_call(
        functools.partial(_k2c_body, B=B, N=N),
        out_shape=(
            jax.ShapeDtypeStruct((B, BW), jnp.int32),
            jax.ShapeDtypeStruct((B, BW), jnp.float32),
            jax.ShapeDtypeStruct((B, BW), jnp.int32),
        ),
    )(g.reshape(B, BW, S), ps, ba)

    return (sel.T.reshape(-1), logp.T.reshape(-1), bbi.T.reshape(-1))


# speedup vs baseline: 4.3836x; 1.0362x over previous
import functools

import jax
import jax.numpy as jnp
from jax import lax
from jax.experimental import pallas as pl
from jax.experimental.pallas import tpu as pltpu
from jax.experimental.pallas import tpu_sc as plsc

BW = 16
S = 400
SW = 512
NEGF = float("-inf")
BIGI = 2**31 - 1


def _k1_body(probs_ref, prev_ref, out_ref, *, P):
    x = probs_ref[...]
    m = jnp.max(x.reshape(x.shape[0], P, S), axis=2)
    out_ref[...] = jnp.log(m) + prev_ref[...]


def _k2a_body(s_ref, prevT_ref, row_ref, col_ref, ps_ref, ba_ref, off_ref, *,
              B, P, N):
    s = s_ref[...]
    wio = lax.broadcasted_iota(jnp.int32, (B, BW, P), 1)
    jio = lax.broadcasted_iota(jnp.int32, (B, BW, P), 2)
    pio = wio * P + jio
    prevT = prevT_ref[...]
    w16 = lax.broadcasted_iota(jnp.int32, (B, BW), 1)
    bcol = lax.broadcasted_iota(jnp.int32, (B, 1), 0)
    ROW = jnp.zeros((B, BW), jnp.int32)
    COL = jnp.zeros((B, BW), jnp.int32)
    PS = jnp.zeros((B, BW), jnp.float32)
    BA = jnp.zeros((B, BW), jnp.int32)
    OFF = jnp.zeros((B, BW), jnp.int32)
    for k in range(BW):
        t = jnp.max(s, axis=2)
        m = jnp.max(t, axis=1, keepdims=True)
        cand = jnp.where(s == m[:, :, None], pio, BIGI)
        c2 = jnp.min(cand, axis=2)
        pid = jnp.min(c2, axis=1, keepdims=True)
        w = pid // P
        j = pid - w * P
        prevk = jnp.sum(jnp.where(w16 == w, prevT, 0.0), axis=1, keepdims=True)
        colm = w16 == k
        ROW = jnp.where(colm, w * 32 + bcol, ROW)
        COL = jnp.where(colm, j * S, COL)
        PS = jnp.where(colm, prevk, PS)
        BA = jnp.where(colm, w * N + j * S, BA)
        OFF = jnp.where(colm, ((w * 32 + bcol) * N + j * S) % 128, OFF)
        s = jnp.where(pio == pid[:, :, None], NEGF, s)
    row_ref[...] = ROW
    col_ref[...] = COL
    ps_ref[...] = PS
    ba_ref[...] = BA
    off_ref[...] = OFF


def _k2c_body(g_ref, ps_ref, ba_ref, off_ref, sel_ref, logp_ref, bbi_ref, *,
              B, N):
    g = g_ref[...]
    cio = lax.broadcasted_iota(jnp.int32, (B, BW, SW), 2)
    off3 = off_ref[...][:, :, None]
    valid = (cio >= off3) & (cio < off3 + S)
    s = jnp.where(valid, jnp.log(g) + ps_ref[...][:, :, None], NEGF)
    gflat = ba_ref[...][:, :, None] + (cio - off3)
    w16 = lax.broadcasted_iota(jnp.int32, (B, BW), 1)
    bcol = lax.broadcasted_iota(jnp.int32, (B, 1), 0)
    SEL = jnp.zeros((B, BW), jnp.int32)
    LOGP = jnp.zeros((B, BW), jnp.float32)
    BBI = jnp.zeros((B, BW), jnp.int32)
    for k in range(BW):
        t = jnp.max(s, axis=2)
        m = jnp.max(t, axis=1, keepdims=True)
        cand = jnp.where(s == m[:, :, None], gflat, BIGI)
        c2 = jnp.min(cand, axis=2)
        wf = jnp.min(c2, axis=1, keepdims=True)
        par = wf // N
        colm = w16 == k
        SEL = jnp.where(colm, wf - par * N, SEL)
        LOGP = jnp.where(colm, m, LOGP)
        BBI = jnp.where(colm, bcol + par * B, BBI)
        s = jnp.where(gflat == wf[:, :, None], NEGF, s)
    sel_ref[...] = SEL
    logp_ref[...] = LOGP
    bbi_ref[...] = BBI


def _gather_body(rows_sm, cols_sm, probs_any, out_ref, sems, *, N):
    b = pl.program_id(0)
    copies = []
    for i in range(BW):
        off = rows_sm[b, i] * N + cols_sm[b, i]
        w0 = pl.multiple_of((off // 128) * 128, 128)
        cp = pltpu.make_async_copy(
            probs_any.at[pl.ds(w0, SW)], out_ref.at[0, i], sems.at[i])
        cp.start()
        copies.append(cp)
    for cp in copies:
        cp.wait()


def _tc_gather(probs, rows, cols):
    B = rows.shape[0]
    N = probs.shape[1]
    return pl.pallas_call(
        functools.partial(_gather_body, N=N),
        grid_spec=pltpu.PrefetchScalarGridSpec(
            num_scalar_prefetch=2,
            grid=(B,),
            in_specs=[pl.BlockSpec(memory_space=pl.ANY)],
            out_specs=pl.BlockSpec((1, BW, SW), lambda b, rows, cols: (b, 0, 0)),
            scratch_shapes=[pltpu.SemaphoreType.DMA((BW,))],
        ),
        out_shape=jax.ShapeDtypeStruct((B, BW, SW), jnp.float32),
    )(rows, cols, probs.reshape(-1))


def kernel(probs, log_beam_prob_prev):
    A, N = probs.shape
    B = A // BW
    P = N // S
    RB = 8

    prev2 = log_beam_prob_prev.reshape(A, 1)

    smax = pl.pallas_call(
        functools.partial(_k1_body, P=P),
        grid=(A // RB,),
        in_specs=[
            pl.BlockSpec((RB, N), lambda i: (i, 0)),
            pl.BlockSpec((RB, 1), lambda i: (i, 0)),
        ],
        out_specs=pl.BlockSpec((RB, P), lambda i: (i, 0)),
        out_shape=jax.ShapeDtypeStruct((A, P), jnp.float32),
    )(probs, prev2)

    smaxT = smax.reshape(BW, B, P).transpose(1, 0, 2)
    prevT = log_beam_prob_prev.reshape(BW, B).T

    row, col, ps, ba, off = pl.pallas_call(
        functools.partial(_k2a_body, B=B, P=P, N=N),
        out_shape=(
            jax.ShapeDtypeStruct((B, BW), jnp.int32),
            jax.ShapeDtypeStruct((B, BW), jnp.int32),
            jax.ShapeDtypeStruct((B, BW), jnp.float32),
            jax.ShapeDtypeStruct((B, BW), jnp.int32),
            jax.ShapeDtypeStruct((B, BW), jnp.int32),
        ),
    )(smaxT, prevT)

    g = _tc_gather(probs, row, col)

    sel, logp, bbi = pl.pallas_call(
        functools.partial(_k2c_body, B=B, N=N),
        out_shape=(
            jax.ShapeDtypeStruct((B, BW), jnp.int32),
            jax.ShapeDtypeStruct((B, BW), jnp.float32),
            jax.ShapeDtypeStruct((B, BW), jnp.int32),
        ),
    )(g, ps, ba, off)

    return (sel.T.reshape(-1), logp.T.reshape(-1), bbi.T.reshape(-1))


# speedup vs baseline: 11.6180x; 2.6503x over previous
import functools

import jax
import jax.numpy as jnp
from jax import lax
from jax.experimental import pallas as pl
from jax.experimental.pallas import tpu as pltpu

BW = 16
S = 400
NEGF = float("-inf")
BIGI = 2**31 - 1


def _k1_body(pt_ref, prev_ref, out_ref, *, PB):
    x = pt_ref[...]
    m = jnp.max(x.reshape(PB, S, x.shape[1]), axis=1)
    out_ref[...] = (jnp.log(m) + prev_ref[...])[None]


def _k2a_body(s_ref, prevT_ref, row_ref, col_ref, ps_ref, ba_ref, *, B, P, N):
    s = s_ref[...]
    wio = lax.broadcasted_iota(jnp.int32, (B, BW, P), 1)
    jio = lax.broadcasted_iota(jnp.int32, (B, BW, P), 2)
    pio = wio * P + jio
    prevT = prevT_ref[...]
    w16 = lax.broadcasted_iota(jnp.int32, (B, BW), 1)
    bcol = lax.broadcasted_iota(jnp.int32, (B, 1), 0)
    ROW = jnp.zeros((B, BW), jnp.int32)
    COL = jnp.zeros((B, BW), jnp.int32)
    PS = jnp.zeros((B, BW), jnp.float32)
    BA = jnp.zeros((B, BW), jnp.int32)
    for k in range(BW):
        t = jnp.max(s, axis=2)
        m = jnp.max(t, axis=1, keepdims=True)
        cand = jnp.where(s == m[:, :, None], pio, BIGI)
        c2 = jnp.min(cand, axis=2)
        pid = jnp.min(c2, axis=1, keepdims=True)
        w = pid // P
        j = pid - w * P
        prevk = jnp.sum(jnp.where(w16 == w, prevT, 0.0), axis=1, keepdims=True)
        colm = w16 == k
        ROW = jnp.where(colm, w * 32 + bcol, ROW)
        COL = jnp.where(colm, j * S, COL)
        PS = jnp.where(colm, prevk, PS)
        BA = jnp.where(colm, w * N + j * S, BA)
        s = jnp.where(pio == pid[:, :, None], NEGF, s)
    row_ref[...] = ROW
    col_ref[...] = COL
    ps_ref[...] = PS
    ba_ref[...] = BA


def _gather_body(rows_sm, cols_sm, pt_any, out_ref, win, sems, *, GPB):
    i = pl.program_id(0)
    copies = []
    for t in range(GPB):
        p = i * GPB + t
        r = rows_sm[p]
        c = pl.multiple_of(cols_sm[p], 8)
        c0 = pl.multiple_of((r // 128) * 128, 128)
        cp = pltpu.make_async_copy(
            pt_any.at[pl.ds(c, S), pl.ds(c0, 128)], win.at[t], sems.at[t])
        cp.start()
        copies.append(cp)
    for cp in copies:
        cp.wait()
    sio = lax.broadcasted_iota(jnp.int32, (128, S), 0)
    for t in range(GPB):
        p = i * GPB + t
        q = rows_sm[p] % 128
        wt = jnp.transpose(win[t], (1, 0))
        rowv = jnp.sum(jnp.where(sio == q, wt, 0.0), axis=0)
        out_ref[t, :] = rowv


def _gather(pt, rows, cols):
    NR = rows.shape[0]
    GPB = 8
    return pl.pallas_call(
        functools.partial(_gather_body, GPB=GPB),
        grid_spec=pltpu.PrefetchScalarGridSpec(
            num_scalar_prefetch=2,
            grid=(NR // GPB,),
            in_specs=[pl.BlockSpec(memory_space=pl.ANY)],
            out_specs=pl.BlockSpec((GPB, S), lambda i, rows, cols: (i, 0)),
            scratch_shapes=[
                pltpu.VMEM((GPB, S, 128), jnp.float32),
                pltpu.SemaphoreType.DMA((GPB,)),
            ],
        ),
        out_shape=jax.ShapeDtypeStruct((NR, S), jnp.float32),
    )(rows, cols, pt)


def _k2c_body(g_ref, ps_ref, ba_ref, sel_ref, logp_ref, bbi_ref, *, B, N):
    g = g_ref[...]
    s = jnp.log(g) + ps_ref[...][:, :, None]
    cio = lax.broadcasted_iota(jnp.int32, (B, BW, S), 2)
    gflat = ba_ref[...][:, :, None] + cio
    w16 = lax.broadcasted_iota(jnp.int32, (B, BW), 1)
    bcol = lax.broadcasted_iota(jnp.int32, (B, 1), 0)
    SEL = jnp.zeros((B, BW), jnp.int32)
    LOGP = jnp.zeros((B, BW), jnp.float32)
    BBI = jnp.zeros((B, BW), jnp.int32)
    for k in range(BW):
        t = jnp.max(s, axis=2)
        m = jnp.max(t, axis=1, keepdims=True)
        cand = jnp.where(s == m[:, :, None], gflat, BIGI)
        c2 = jnp.min(cand, axis=2)
        wf = jnp.min(c2, axis=1, keepdims=True)
        par = wf // N
        colm = w16 == k
        SEL = jnp.where(colm, wf - par * N, SEL)
        LOGP = jnp.where(colm, m, LOGP)
        BBI = jnp.where(colm, bcol + par * B, BBI)
        s = jnp.where(gflat == wf[:, :, None], NEGF, s)
    sel_ref[...] = SEL
    logp_ref[...] = LOGP
    bbi_ref[...] = BBI


def kernel(probs, log_beam_prob_prev):
    A, N = probs.shape
    B = A // BW
    P = N // S
    PB = 5
    pt = probs.T

    prev1 = log_beam_prob_prev.reshape(1, A)

    smaxc = pl.pallas_call(
        functools.partial(_k1_body, PB=PB),
        grid=(P // PB,),
        in_specs=[
            pl.BlockSpec((PB * S, A), lambda i: (i, 0)),
            pl.BlockSpec((1, A), lambda i: (0, 0)),
        ],
        out_specs=pl.BlockSpec((1, PB, A), lambda i: (i, 0, 0)),
        out_shape=jax.ShapeDtypeStruct((P // PB, PB, A), jnp.float32),
    )(pt, prev1)

    smaxT = smaxc.reshape(P, BW, B).transpose(2, 1, 0)
    prevT = log_beam_prob_prev.reshape(BW, B).T

    row, col, ps, ba = pl.pallas_call(
        functools.partial(_k2a_body, B=B, P=P, N=N),
        out_shape=(
            jax.ShapeDtypeStruct((B, BW), jnp.int32),
            jax.ShapeDtypeStruct((B, BW), jnp.int32),
            jax.ShapeDtypeStruct((B, BW), jnp.float32),
            jax.ShapeDtypeStruct((B, BW), jnp.int32),
        ),
    )(smaxT, prevT)

    g = _gather(pt, row.reshape(-1), col.reshape(-1))

    sel, logp, bbi = pl.pallas_call(
        functools.partial(_k2c_body, B=B, N=N),
        out_shape=(
            jax.ShapeDtypeStruct((B, BW), jnp.int32),
            jax.ShapeDtypeStruct((B, BW), jnp.float32),
            jax.ShapeDtypeStruct((B, BW), jnp.int32),
        ),
    )(g.reshape(B, BW, S), ps, ba)

    return (sel.T.reshape(-1), logp.T.reshape(-1), bbi.T.reshape(-1))


# speedup vs baseline: 17.5282x; 1.5087x over previous
import functools

import jax
import jax.numpy as jnp
from jax import lax
from jax.experimental import pallas as pl
from jax.experimental.pallas import tpu as pltpu

BW = 16
S = 200
NEGF = float("-inf")
BIGI = 2**31 - 1


def _k1_body(pt_ref, prev_ref, out_ref, *, PB):
    x = pt_ref[...]
    m = jnp.max(x.reshape(PB, S, x.shape[1]), axis=1)
    out_ref[...] = (jnp.log(m) + prev_ref[...])[None]


def _k2a_body(s_ref, prevT_ref, row_ref, col_ref, ps_ref, ba_ref, *, B, P, N):
    s = s_ref[...]
    wio = lax.broadcasted_iota(jnp.int32, (B, BW, P), 1)
    jio = lax.broadcasted_iota(jnp.int32, (B, BW, P), 2)
    pio = wio * P + jio
    prevT = prevT_ref[...]
    w16 = lax.broadcasted_iota(jnp.int32, (B, BW), 1)
    bcol = lax.broadcasted_iota(jnp.int32, (B, 1), 0)
    ROW = jnp.zeros((B, BW), jnp.int32)
    COL = jnp.zeros((B, BW), jnp.int32)
    PS = jnp.zeros((B, BW), jnp.float32)
    BA = jnp.zeros((B, BW), jnp.int32)
    for k in range(BW):
        t = jnp.max(s, axis=2)
        m = jnp.max(t, axis=1, keepdims=True)
        cand = jnp.where(s == m[:, :, None], pio, BIGI)
        c2 = jnp.min(cand, axis=2)
        pid = jnp.min(c2, axis=1, keepdims=True)
        w = pid // P
        j = pid - w * P
        prevk = jnp.sum(jnp.where(w16 == w, prevT, 0.0), axis=1, keepdims=True)
        colm = w16 == k
        ROW = jnp.where(colm, w * 32 + bcol, ROW)
        COL = jnp.where(colm, j * S, COL)
        PS = jnp.where(colm, prevk, PS)
        BA = jnp.where(colm, w * N + j * S, BA)
        s = jnp.where(pio == pid[:, :, None], NEGF, s)
    row_ref[...] = ROW
    col_ref[...] = COL
    ps_ref[...] = PS
    ba_ref[...] = BA


def _gather_body(rows_sm, cols_sm, pt_any, out_ref, win, sems, *, GPB, NG):
    i = pl.program_id(0)

    def issue(group, buf):
        for t in range(GPB):
            p = group * GPB + t
            r = rows_sm[p]
            c = pl.multiple_of(cols_sm[p], 8)
            c0 = pl.multiple_of((r // 128) * 128, 128)
            pltpu.make_async_copy(
                pt_any.at[pl.ds(c, S), pl.ds(c0, 128)],
                win.at[buf, t], sems.at[buf, t]).start()

    @pl.when(i == 0)
    def _():
        issue(0, 0)

    @pl.when(i + 1 < NG)
    def _():
        issue(i + 1, (i + 1) % 2)

    buf = i % 2
    for t in range(GPB):
        pltpu.make_async_copy(
            pt_any.at[pl.ds(0, S), pl.ds(0, 128)],
            win.at[buf, t], sems.at[buf, t]).wait()
    sio = lax.broadcasted_iota(jnp.int32, (128, S), 0)
    for t in range(GPB):
        p = i * GPB + t
        q = rows_sm[p] % 128
        wt = jnp.transpose(win[buf, t], (1, 0))
        rowv = jnp.sum(jnp.where(sio == q, wt, 0.0), axis=0)
        out_ref[t, :] = rowv


def _gather(pt, rows, cols):
    NR = rows.shape[0]
    GPB = 8
    NG = NR // GPB
    return pl.pallas_call(
        functools.partial(_gather_body, GPB=GPB, NG=NG),
        grid_spec=pltpu.PrefetchScalarGridSpec(
            num_scalar_prefetch=2,
            grid=(NG,),
            in_specs=[pl.BlockSpec(memory_space=pl.ANY)],
            out_specs=pl.BlockSpec((GPB, S), lambda i, rows, cols: (i, 0)),
            scratch_shapes=[
                pltpu.VMEM((2, GPB, S, 128), jnp.float32),
                pltpu.SemaphoreType.DMA((2, GPB)),
            ],
        ),
        out_shape=jax.ShapeDtypeStruct((NR, S), jnp.float32),
    )(rows, cols, pt)


def _k2c_body(g_ref, ps_ref, ba_ref, sel_ref, logp_ref, bbi_ref, *, B, N):
    g = g_ref[...]
    s = jnp.log(g) + ps_ref[...][:, :, None]
    cio = lax.broadcasted_iota(jnp.int32, (B, BW, S), 2)
    gflat = ba_ref[...][:, :, None] + cio
    w16 = lax.broadcasted_iota(jnp.int32, (B, BW), 1)
    bcol = lax.broadcasted_iota(jnp.int32, (B, 1), 0)
    SEL = jnp.zeros((B, BW), jnp.int32)
    LOGP = jnp.zeros((B, BW), jnp.float32)
    BBI = jnp.zeros((B, BW), jnp.int32)
    for k in range(BW):
        t = jnp.max(s, axis=2)
        m = jnp.max(t, axis=1, keepdims=True)
        cand = jnp.where(s == m[:, :, None], gflat, BIGI)
        c2 = jnp.min(cand, axis=2)
        wf = jnp.min(c2, axis=1, keepdims=True)
        par = wf // N
        colm = w16 == k
        SEL = jnp.where(colm, wf - par * N, SEL)
        LOGP = jnp.where(colm, m, LOGP)
        BBI = jnp.where(colm, bcol + par * B, BBI)
        s = jnp.where(gflat == wf[:, :, None], NEGF, s)
    sel_ref[...] = SEL
    logp_ref[...] = LOGP
    bbi_ref[...] = BBI


def kernel(probs, log_beam_prob_prev):
    A, N = probs.shape
    B = A // BW
    P = N // S
    PB = 10
    pt = probs.T

    prev1 = log_beam_prob_prev.reshape(1, A)

    smaxc = pl.pallas_call(
        functools.partial(_k1_body, PB=PB),
        grid=(P // PB,),
        in_specs=[
            pl.BlockSpec((PB * S, A), lambda i: (i, 0)),
            pl.BlockSpec((1, A), lambda i: (0, 0)),
        ],
        out_specs=pl.BlockSpec((1, PB, A), lambda i: (i, 0, 0)),
        out_shape=jax.ShapeDtypeStruct((P // PB, PB, A), jnp.float32),
    )(pt, prev1)

    smaxT = smaxc.reshape(P, BW, B).transpose(2, 1, 0)
    prevT = log_beam_prob_prev.reshape(BW, B).T

    row, col, ps, ba = pl.pallas_call(
        functools.partial(_k2a_body, B=B, P=P, N=N),
        out_shape=(
            jax.ShapeDtypeStruct((B, BW), jnp.int32),
            jax.ShapeDtypeStruct((B, BW), jnp.int32),
            jax.ShapeDtypeStruct((B, BW), jnp.float32),
            jax.ShapeDtypeStruct((B, BW), jnp.int32),
        ),
    )(smaxT, prevT)

    g = _gather(pt, row.reshape(-1), col.reshape(-1))

    sel, logp, bbi = pl.pallas_call(
        functools.partial(_k2c_body, B=B, N=N),
        out_shape=(
            jax.ShapeDtypeStruct((B, BW), jnp.int32),
            jax.ShapeDtypeStruct((B, BW), jnp.float32),
            jax.ShapeDtypeStruct((B, BW), jnp.int32),
        ),
    )(g.reshape(B, BW, S), ps, ba)

    return (sel.T.reshape(-1), logp.T.reshape(-1), bbi.T.reshape(-1))


# speedup vs baseline: 19.4126x; 1.1075x over previous
import functools

import jax
import jax.numpy as jnp
from jax import lax
from jax.experimental import pallas as pl
from jax.experimental.pallas import tpu as pltpu

BW = 16
S = 200
NEGF = float("-inf")
BIGI = 2**31 - 1


def _k1_body(pt_ref, prev_ref, out_ref, *, PB):
    x = pt_ref[...]
    m = jnp.max(x.reshape(PB, S, x.shape[1]), axis=1)
    out_ref[...] = (jnp.log(m) + prev_ref[...])[None]


def _k2a_body(s_ref, prevT_ref, row_ref, col_ref, ps_ref, ba_ref, *, B, P, N):
    s = s_ref[...]
    wio = lax.broadcasted_iota(jnp.int32, (B, BW, P), 1)
    jio = lax.broadcasted_iota(jnp.int32, (B, BW, P), 2)
    pio = wio * P + jio
    prevT = prevT_ref[...]
    w16 = lax.broadcasted_iota(jnp.int32, (B, BW), 1)
    bcol = lax.broadcasted_iota(jnp.int32, (B, 1), 0)
    ROW = jnp.zeros((B, BW), jnp.int32)
    COL = jnp.zeros((B, BW), jnp.int32)
    PS = jnp.zeros((B, BW), jnp.float32)
    BA = jnp.zeros((B, BW), jnp.int32)
    for k in range(BW):
        t = jnp.max(s, axis=2)
        m = jnp.max(t, axis=1, keepdims=True)
        cand = jnp.where(s == m[:, :, None], pio, BIGI)
        c2 = jnp.min(cand, axis=2)
        pid = jnp.min(c2, axis=1, keepdims=True)
        w = pid // P
        j = pid - w * P
        prevk = jnp.sum(jnp.where(w16 == w, prevT, 0.0), axis=1, keepdims=True)
        colm = w16 == k
        ROW = jnp.where(colm, w * 32 + bcol, ROW)
        COL = jnp.where(colm, j * S, COL)
        PS = jnp.where(colm, prevk, PS)
        BA = jnp.where(colm, w * N + j * S, BA)
        s = jnp.where(pio == pid[:, :, None], NEGF, s)
    row_ref[...] = ROW
    col_ref[...] = COL
    ps_ref[...] = PS
    ba_ref[...] = BA


def _gather_body(rows_sm, cols_sm, pt_any, out_ref, win, sems, *, GPB, NG):
    i = pl.program_id(0)

    def issue(group, buf):
        for t in range(GPB):
            p = group * GPB + t
            r = rows_sm[p]
            c = pl.multiple_of(cols_sm[p], 8)
            c0 = pl.multiple_of((r // 128) * 128, 128)
            pltpu.make_async_copy(
                pt_any.at[pl.ds(c, S), pl.ds(c0, 128)],
                win.at[buf, t], sems.at[buf, t]).start()

    @pl.when(i == 0)
    def _():
        issue(0, 0)

    @pl.when(i + 1 < NG)
    def _():
        issue(i + 1, (i + 1) % 2)

    buf = i % 2
    for t in range(GPB):
        pltpu.make_async_copy(
            pt_any.at[pl.ds(0, S), pl.ds(0, 128)],
            win.at[buf, t], sems.at[buf, t]).wait()
    sio = lax.broadcasted_iota(jnp.int32, (128, S), 0)
    for t in range(GPB):
        p = i * GPB + t
        q = rows_sm[p] % 128
        wt = jnp.transpose(win[buf, t], (1, 0))
        rowv = jnp.sum(jnp.where(sio == q, wt, 0.0), axis=0)
        out_ref[t, :] = rowv


def _gather(pt, rows, cols):
    NR = rows.shape[0]
    GPB = 16
    NG = NR // GPB
    return pl.pallas_call(
        functools.partial(_gather_body, GPB=GPB, NG=NG),
        grid_spec=pltpu.PrefetchScalarGridSpec(
            num_scalar_prefetch=2,
            grid=(NG,),
            in_specs=[pl.BlockSpec(memory_space=pl.ANY)],
            out_specs=pl.BlockSpec((GPB, S), lambda i, rows, cols: (i, 0)),
            scratch_shapes=[
                pltpu.VMEM((2, GPB, S, 128), jnp.float32),
                pltpu.SemaphoreType.DMA((2, GPB)),
            ],
        ),
        out_shape=jax.ShapeDtypeStruct((NR, S), jnp.float32),
    )(rows, cols, pt)


def _k2c_body(g_ref, ps_ref, ba_ref, sel_ref, logp_ref, bbi_ref, *, B, N):
    g = g_ref[...]
    s = jnp.log(g) + ps_ref[...][:, :, None]
    cio = lax.broadcasted_iota(jnp.int32, (B, BW, S), 2)
    gflat = ba_ref[...][:, :, None] + cio
    w16 = lax.broadcasted_iota(jnp.int32, (B, BW), 1)
    bcol = lax.broadcasted_iota(jnp.int32, (B, 1), 0)
    SEL = jnp.zeros((B, BW), jnp.int32)
    LOGP = jnp.zeros((B, BW), jnp.float32)
    BBI = jnp.zeros((B, BW), jnp.int32)
    for k in range(BW):
        t = jnp.max(s, axis=2)
        m = jnp.max(t, axis=1, keepdims=True)
        cand = jnp.where(s == m[:, :, None], gflat, BIGI)
        c2 = jnp.min(cand, axis=2)
        wf = jnp.min(c2, axis=1, keepdims=True)
        par = wf // N
        colm = w16 == k
        SEL = jnp.where(colm, wf - par * N, SEL)
        LOGP = jnp.where(colm, m, LOGP)
        BBI = jnp.where(colm, bcol + par * B, BBI)
        s = jnp.where(gflat == wf[:, :, None], NEGF, s)
    sel_ref[...] = SEL
    logp_ref[...] = LOGP
    bbi_ref[...] = BBI


def kernel(probs, log_beam_prob_prev):
    A, N = probs.shape
    B = A // BW
    P = N // S
    PB = 10
    pt = probs.T

    prev1 = log_beam_prob_prev.reshape(1, A)

    smaxc = pl.pallas_call(
        functools.partial(_k1_body, PB=PB),
        grid=(P // PB,),
        in_specs=[
            pl.BlockSpec((PB * S, A), lambda i: (i, 0)),
            pl.BlockSpec((1, A), lambda i: (0, 0)),
        ],
        out_specs=pl.BlockSpec((1, PB, A), lambda i: (i, 0, 0)),
        out_shape=jax.ShapeDtypeStruct((P // PB, PB, A), jnp.float32),
    )(pt, prev1)

    smaxT = smaxc.reshape(P, BW, B).transpose(2, 1, 0)
    prevT = log_beam_prob_prev.reshape(BW, B).T

    row, col, ps, ba = pl.pallas_call(
        functools.partial(_k2a_body, B=B, P=P, N=N),
        out_shape=(
            jax.ShapeDtypeStruct((B, BW), jnp.int32),
            jax.ShapeDtypeStruct((B, BW), jnp.int32),
            jax.ShapeDtypeStruct((B, BW), jnp.float32),
            jax.ShapeDtypeStruct((B, BW), jnp.int32),
        ),
    )(smaxT, prevT)

    g = _gather(pt, row.reshape(-1), col.reshape(-1))

    sel, logp, bbi = pl.pallas_call(
        functools.partial(_k2c_body, B=B, N=N),
        out_shape=(
            jax.ShapeDtypeStruct((B, BW), jnp.int32),
            jax.ShapeDtypeStruct((B, BW), jnp.float32),
            jax.ShapeDtypeStruct((B, BW), jnp.int32),
        ),
    )(g.reshape(B, BW, S), ps, ba)

    return (sel.T.reshape(-1), logp.T.reshape(-1), bbi.T.reshape(-1))
